# baseline (speedup 1.0000x reference)
import functools

import jax
import jax.numpy as jnp
from jax import lax
from jax.experimental import pallas as pl
from jax.experimental.pallas import tpu as pltpu
from jax.experimental.pallas import tpu_sc as plsc

DIM = 2048
N_EXPERTS = 64
N_GROUPS = 8
GROUP_SIZE = N_EXPERTS // N_GROUPS
TOPK_GROUPS = 4
TOPK = 8
ROUTE_SCALE = 2.5
N_TOK = 16384

BT = 512

L = 16
NW = 32
N_CHUNKS = 4
TOKC = N_TOK // N_CHUNKS
TOK_PER_W = TOKC // NW


NE_PAD = 128


def _score_kernel(x_ref, wt_ref, s_ref):
    s_ref[...] = jax.nn.sigmoid(
        jnp.dot(x_ref[...], wt_ref[...], preferred_element_type=jnp.float32)
    )


def _tree_max(vs):
    while len(vs) > 1:
        nxt = [jnp.maximum(vs[i], vs[i + 1]) for i in range(0, len(vs) - 1, 2)]
        if len(vs) % 2:
            nxt.append(vs[-1])
        vs = nxt
    return vs[0]


def _route_kernel(s_hbm, w_hbm, i_hbm, s_v, w_v, i_v):
    wid = lax.axis_index("s") * 2 + lax.axis_index("c")
    base = wid * TOK_PER_W
    pltpu.sync_copy(s_hbm.at[pl.ds(base * NE_PAD, TOK_PER_W * NE_PAD)], s_v)

    lanes = lax.iota(jnp.int32, L)

    def body(i, carry):
        tok = i * L + lanes

        tok64 = tok * NE_PAD
        s = [
            plsc.load_gather(s_v, [tok64 + e])
            for e in range(N_EXPERTS)
        ]

        gm = [
            _tree_max(s[g * GROUP_SIZE:(g + 1) * GROUP_SIZE])
            for g in range(N_GROUPS)
        ]

        neg1 = jnp.full((L,), -1.0, jnp.float32)
        gsel8 = []
        work = list(gm)
        for _q in range(TOPK_GROUPS):
            cur = _tree_max(work)
            gidx = jnp.full((L,), N_GROUPS, jnp.int32)
            for g in range(N_GROUPS):
                gidx = jnp.minimum(
                    gidx,
                    jnp.where(work[g] == cur,
                              jnp.full((L,), g, jnp.int32),
                              jnp.full((L,), N_GROUPS, jnp.int32)),
                )
            for g in range(N_GROUPS):
                work[g] = jnp.where(gidx == g, neg1, work[g])
            gsel8.append(gidx * GROUP_SIZE)

        cands = []
        cols = []
        for q in range(TOPK_GROUPS):
            for j in range(GROUP_SIZE):
                col = gsel8[q] + j
                cols.append(col)
                cands.append(plsc.load_gather(s_v, [tok64 + col]))

        big = jnp.full((L,), N_EXPERTS, jnp.int32)
        ws = []
        idxs = []
        for _k in range(TOPK):
            cur = _tree_max(cands)
            idx = big
            for e in range(len(cands)):
                idx = jnp.minimum(idx, jnp.where(cands[e] == cur, cols[e], big))
            for e in range(len(cands)):
                cands[e] = jnp.where(cols[e] == idx, neg1, cands[e])
            ws.append(cur)
            idxs.append(idx)

        total = (ws[0] + ws[1]) + (ws[2] + ws[3]) + ((ws[4] + ws[5])
                                                    + (ws[6] + ws[7]))
        scale = ROUTE_SCALE / total
        tok8 = tok * TOPK
        for k in range(TOPK):
            plsc.store_scatter(w_v, [tok8 + k], ws[k] * scale)
            plsc.store_scatter(i_v, [tok8 + k], idxs[k])
        return carry

    lax.fori_loop(0, TOK_PER_W // L, body, 0)

    pltpu.sync_copy(w_v, w_hbm.at[pl.ds(base * TOPK, TOK_PER_W * TOPK)])
    pltpu.sync_copy(i_v, i_hbm.at[pl.ds(base * TOPK, TOK_PER_W * TOPK)])


_route = functools.partial(
    pl.kernel,
    mesh=plsc.VectorSubcoreMesh(core_axis_name="c", subcore_axis_name="s"),
    out_type=[
        jax.ShapeDtypeStruct((TOKC * TOPK,), jnp.float32),
        jax.ShapeDtypeStruct((TOKC * TOPK,), jnp.int32),
    ],
    scratch_types=[
        pltpu.VMEM((TOK_PER_W * NE_PAD,), jnp.float32),
        pltpu.VMEM((TOK_PER_W * TOPK,), jnp.float32),
        pltpu.VMEM((TOK_PER_W * TOPK,), jnp.int32),
    ],
    compiler_params=pltpu.CompilerParams(needs_layout_passes=False),
)(_route_kernel)


@jax.jit
def kernel(x, weight):
    n = x.shape[0]
    wt = jnp.zeros((DIM, NE_PAD), jnp.float32).at[:, :N_EXPERTS].set(weight.T)
    w_parts = []
    i_parts = []
    for c in range(N_CHUNKS):
        xc = lax.slice_in_dim(x, c * TOKC, (c + 1) * TOKC, axis=0)
        scores = pl.pallas_call(
            _score_kernel,
            grid=(TOKC // BT,),
            in_specs=[
                pl.BlockSpec((BT, DIM), lambda i: (i, 0)),
                pl.BlockSpec((DIM, NE_PAD), lambda i: (0, 0)),
            ],
            out_specs=pl.BlockSpec((BT, NE_PAD), lambda i: (i, 0)),
            out_shape=jax.ShapeDtypeStruct((TOKC, NE_PAD), jnp.float32),
        )(xc, wt)
        w_flat, i_flat = _route(scores.reshape(-1))
        w_parts.append(w_flat.reshape(TOKC, TOPK))
        i_parts.append(i_flat.reshape(TOKC, TOPK))
    return (jnp.concatenate(w_parts, axis=0),
            jnp.concatenate(i_parts, axis=0))


# speedup vs baseline: 1.7930x; 1.7930x over previous
import functools

import jax
import jax.numpy as jnp
from jax import lax
from jax.experimental import pallas as pl
from jax.experimental.pallas import tpu as pltpu
from jax.experimental.pallas import tpu_sc as plsc

DIM = 2048
N_EXPERTS = 64
N_GROUPS = 8
GROUP_SIZE = N_EXPERTS // N_GROUPS
TOPK_GROUPS = 4
TOPK = 8
ROUTE_SCALE = 2.5
N_TOK = 16384

BT = 512

L = 16
NW = 32
N_CHUNKS = 4
TOKC = N_TOK // N_CHUNKS
TOK_PER_W = TOKC // NW


NE_PAD = 128


def _score_kernel(x_ref, wt_ref, s_ref):
    s_ref[...] = jax.nn.sigmoid(
        jnp.dot(x_ref[...], wt_ref[...], preferred_element_type=jnp.float32)
    )


def _tree_max(vs):
    while len(vs) > 1:
        nxt = [jnp.maximum(vs[i], vs[i + 1]) for i in range(0, len(vs) - 1, 2)]
        if len(vs) % 2:
            nxt.append(vs[-1])
        vs = nxt
    return vs[0]


def _route_kernel(s_hbm, w_hbm, i_hbm, s_v, w_v, i_v):
    wid = lax.axis_index("s") * 2 + lax.axis_index("c")
    base = wid * TOK_PER_W
    pltpu.sync_copy(s_hbm.at[pl.ds(base * NE_PAD, TOK_PER_W * NE_PAD)], s_v)

    lanes = lax.iota(jnp.int32, L)

    def body(i, carry):
        tok = i * L + lanes

        tok64 = tok * NE_PAD
        s = [
            plsc.load_gather(s_v, [tok64 + e])
            for e in range(N_EXPERTS)
        ]

        gm = [
            _tree_max(s[g * GROUP_SIZE:(g + 1) * GROUP_SIZE])
            for g in range(N_GROUPS)
        ]

        neg1 = jnp.full((L,), -1.0, jnp.float32)
        gsel8 = []
        work = list(gm)
        for _q in range(TOPK_GROUPS):
            cur = _tree_max(work)
            gidx = jnp.full((L,), N_GROUPS, jnp.int32)
            for g in range(N_GROUPS):
                gidx = jnp.minimum(
                    gidx,
                    jnp.where(work[g] == cur,
                              jnp.full((L,), g, jnp.int32),
                              jnp.full((L,), N_GROUPS, jnp.int32)),
                )
            for g in range(N_GROUPS):
                work[g] = jnp.where(gidx == g, neg1, work[g])
            gsel8.append(gidx * GROUP_SIZE)

        cands = []
        cols = []
        for q in range(TOPK_GROUPS):
            for j in range(GROUP_SIZE):
                col = gsel8[q] + j
                cols.append(col)
                cands.append(plsc.load_gather(s_v, [tok64 + col]))

        big = jnp.full((L,), N_EXPERTS, jnp.int32)
        ws = []
        idxs = []
        for _k in range(TOPK):
            cur = _tree_max(cands)
            idx = big
            for e in range(len(cands)):
                idx = jnp.minimum(idx, jnp.where(cands[e] == cur, cols[e], big))
            for e in range(len(cands)):
                cands[e] = jnp.where(cols[e] == idx, neg1, cands[e])
            ws.append(cur)
            idxs.append(idx)

        total = (ws[0] + ws[1]) + (ws[2] + ws[3]) + ((ws[4] + ws[5])
                                                    + (ws[6] + ws[7]))
        scale = ROUTE_SCALE / total
        tok8 = tok * TOPK
        for k in range(TOPK):
            plsc.store_scatter(w_v, [tok8 + k], ws[k] * scale)
            plsc.store_scatter(i_v, [tok8 + k], idxs[k])
        return carry

    lax.fori_loop(0, TOK_PER_W // L, body, 0)

    pltpu.sync_copy(w_v, w_hbm.at[pl.ds(base * TOPK, TOK_PER_W * TOPK)])
    pltpu.sync_copy(i_v, i_hbm.at[pl.ds(base * TOPK, TOK_PER_W * TOPK)])


_route = functools.partial(
    pl.kernel,
    mesh=plsc.VectorSubcoreMesh(core_axis_name="c", subcore_axis_name="s"),
    out_type=[
        jax.ShapeDtypeStruct((TOKC * TOPK,), jnp.float32),
        jax.ShapeDtypeStruct((TOKC * TOPK,), jnp.int32),
    ],
    scratch_types=[
        pltpu.VMEM((TOK_PER_W * NE_PAD,), jnp.float32),
        pltpu.VMEM((TOK_PER_W * TOPK,), jnp.float32),
        pltpu.VMEM((TOK_PER_W * TOPK,), jnp.int32),
    ],
    compiler_params=pltpu.CompilerParams(needs_layout_passes=False),
)(_route_kernel)


@jax.jit
def kernel(x, weight):
    n = x.shape[0]
    wt = jnp.zeros((DIM, NE_PAD), jnp.float32).at[:, :N_EXPERTS].set(weight.T)
    w_parts = []
    i_parts = []
    blocks_per_chunk = TOKC // BT
    for c in range(N_CHUNKS):
        off = c * blocks_per_chunk
        scores = pl.pallas_call(
            _score_kernel,
            grid=(blocks_per_chunk,),
            in_specs=[
                pl.BlockSpec((BT, DIM), lambda i, off=off: (off + i, 0)),
                pl.BlockSpec((DIM, NE_PAD), lambda i: (0, 0)),
            ],
            out_specs=pl.BlockSpec((BT, NE_PAD), lambda i: (i, 0)),
            out_shape=jax.ShapeDtypeStruct((TOKC, NE_PAD), jnp.float32),
        )(x, wt)
        w_flat, i_flat = _route(scores.reshape(-1))
        w_parts.append(w_flat.reshape(TOKC, TOPK))
        i_parts.append(i_flat.reshape(TOKC, TOPK))
    return (jnp.concatenate(w_parts, axis=0),
            jnp.concatenate(i_parts, axis=0))


# speedup vs baseline: 1.8999x; 1.0596x over previous
import functools

import jax
import jax.numpy as jnp
from jax import lax
from jax.experimental import pallas as pl
from jax.experimental.pallas import tpu as pltpu
from jax.experimental.pallas import tpu_sc as plsc

DIM = 2048
N_EXPERTS = 64
N_GROUPS = 8
GROUP_SIZE = N_EXPERTS // N_GROUPS
TOPK_GROUPS = 4
TOPK = 8
ROUTE_SCALE = 2.5
N_TOK = 16384

BT = 512

L = 16
NW = 32
N_CHUNKS = 4
TOKC = N_TOK // N_CHUNKS
TOK_PER_W = TOKC // NW


NE_PAD = 128


def _score_kernel(x_ref, w_ref, s_ref):
    scores = jax.nn.sigmoid(
        jax.lax.dot_general(
            x_ref[...], w_ref[...],
            dimension_numbers=(((1,), (1,)), ((), ())),
            preferred_element_type=jnp.float32,
        )
    )
    pad = jnp.zeros((scores.shape[0], NE_PAD - N_EXPERTS), jnp.float32)
    s_ref[...] = jnp.concatenate([scores, pad], axis=1)


def _tree_max(vs):
    while len(vs) > 1:
        nxt = [jnp.maximum(vs[i], vs[i + 1]) for i in range(0, len(vs) - 1, 2)]
        if len(vs) % 2:
            nxt.append(vs[-1])
        vs = nxt
    return vs[0]


def _route_kernel(s_hbm, w_hbm, i_hbm, s_v, w_v, i_v):
    wid = lax.axis_index("s") * 2 + lax.axis_index("c")
    base = wid * TOK_PER_W
    pltpu.sync_copy(s_hbm.at[pl.ds(base * NE_PAD, TOK_PER_W * NE_PAD)], s_v)

    lanes = lax.iota(jnp.int32, L)

    def body(i, carry):
        tok = i * L + lanes

        tok64 = tok * NE_PAD
        s = [
            plsc.load_gather(s_v, [tok64 + e])
            for e in range(N_EXPERTS)
        ]

        gm = [
            _tree_max(s[g * GROUP_SIZE:(g + 1) * GROUP_SIZE])
            for g in range(N_GROUPS)
        ]

        neg1 = jnp.full((L,), -1.0, jnp.float32)
        gsel8 = []
        work = list(gm)
        for _q in range(TOPK_GROUPS):
            cur = _tree_max(work)
            gidx = jnp.full((L,), N_GROUPS, jnp.int32)
            for g in range(N_GROUPS):
                gidx = jnp.minimum(
                    gidx,
                    jnp.where(work[g] == cur,
                              jnp.full((L,), g, jnp.int32),
                              jnp.full((L,), N_GROUPS, jnp.int32)),
                )
            for g in range(N_GROUPS):
                work[g] = jnp.where(gidx == g, neg1, work[g])
            gsel8.append(gidx * GROUP_SIZE)

        cands = []
        cols = []
        for q in range(TOPK_GROUPS):
            for j in range(GROUP_SIZE):
                col = gsel8[q] + j
                cols.append(col)
                cands.append(plsc.load_gather(s_v, [tok64 + col]))

        big = jnp.full((L,), N_EXPERTS, jnp.int32)
        ws = []
        idxs = []
        for _k in range(TOPK):
            cur = _tree_max(cands)
            idx = big
            for e in range(len(cands)):
                idx = jnp.minimum(idx, jnp.where(cands[e] == cur, cols[e], big))
            for e in range(len(cands)):
                cands[e] = jnp.where(cols[e] == idx, neg1, cands[e])
            ws.append(cur)
            idxs.append(idx)

        total = (ws[0] + ws[1]) + (ws[2] + ws[3]) + ((ws[4] + ws[5])
                                                    + (ws[6] + ws[7]))
        scale = ROUTE_SCALE / total
        tok8 = tok * TOPK
        for k in range(TOPK):
            plsc.store_scatter(w_v, [tok8 + k], ws[k] * scale)
            plsc.store_scatter(i_v, [tok8 + k], idxs[k])
        return carry

    lax.fori_loop(0, TOK_PER_W // L, body, 0)

    pltpu.sync_copy(w_v, w_hbm.at[pl.ds(base * TOPK, TOK_PER_W * TOPK)])
    pltpu.sync_copy(i_v, i_hbm.at[pl.ds(base * TOPK, TOK_PER_W * TOPK)])


_route = functools.partial(
    pl.kernel,
    mesh=plsc.VectorSubcoreMesh(core_axis_name="c", subcore_axis_name="s"),
    out_type=[
        jax.ShapeDtypeStruct((TOKC * TOPK,), jnp.float32),
        jax.ShapeDtypeStruct((TOKC * TOPK,), jnp.int32),
    ],
    scratch_types=[
        pltpu.VMEM((TOK_PER_W * NE_PAD,), jnp.float32),
        pltpu.VMEM((TOK_PER_W * TOPK,), jnp.float32),
        pltpu.VMEM((TOK_PER_W * TOPK,), jnp.int32),
    ],
    compiler_params=pltpu.CompilerParams(needs_layout_passes=False),
)(_route_kernel)


@jax.jit
def kernel(x, weight):
    n = x.shape[0]
    w_parts = []
    i_parts = []
    for c in range(N_CHUNKS):
        off = c * (TOKC // BT)
        scores = pl.pallas_call(
            _score_kernel,
            grid=(TOKC // BT,),
            in_specs=[
                pl.BlockSpec((BT, DIM), lambda i, off=off: (off + i, 0)),
                pl.BlockSpec((N_EXPERTS, DIM), lambda i: (0, 0)),
            ],
            out_specs=pl.BlockSpec((BT, NE_PAD), lambda i: (i, 0)),
            out_shape=jax.ShapeDtypeStruct((TOKC, NE_PAD), jnp.float32),
        )(x, weight)
        w_flat, i_flat = _route(scores.reshape(-1))
        w_parts.append(w_flat.reshape(TOKC, TOPK))
        i_parts.append(i_flat.reshape(TOKC, TOPK))
    return (jnp.concatenate(w_parts, axis=0),
            jnp.concatenate(i_parts, axis=0))
